# Initial kernel scaffold; baseline (speedup 1.0000x reference)
import jax
import jax.numpy as jnp
from jax.experimental import pallas as pl


def kernel(types, table):
    raise NotImplementedError("write your pallas kernel here")


# speedup vs baseline: 1.6208x; 1.6208x over previous
import functools

import jax
import jax.numpy as jnp
from jax import lax
from jax.experimental import pallas as pl
from jax.experimental.pallas import tpu as pltpu
from jax.experimental.pallas import tpu_sc as plsc

_BATCH = 16384
_HIST = 200
_DIM = 64
_NT = 4
_BP = _BATCH * _HIST // 2
_PD = 2 * _DIM
_NC = 2
_NS = 16
_NW = _NC * _NS
_RPG = 128
_KC = 4
_CHUNK = _KC * _RPG
_B_PER_W = _BP // _NW
_N_CHUNKS = _B_PER_W // _CHUNK

_mesh = plsc.VectorSubcoreMesh(core_axis_name="c", subcore_axis_name="s")


@functools.partial(
    pl.kernel,
    out_type=jax.ShapeDtypeStruct((_BP, _PD), jnp.float32),
    mesh=_mesh,
    scratch_types=[
        pltpu.VMEM((_KC, _RPG), jnp.int32),
        pltpu.VMEM((_CHUNK, _PD), jnp.float32),
        pltpu.SemaphoreType.DMA,
    ],
)
def _embed_sc(idx_hbm, ptable_hbm, out_hbm, idx_v, rows_v, sem):
    wid = lax.axis_index("s") * _NC + lax.axis_index("c")
    row0 = wid * (_B_PER_W // _RPG)

    def body(c, carry):
        base = row0 + c * _KC
        pltpu.sync_copy(idx_hbm.at[pl.ds(base, _KC)], idx_v)
        copies = [
            pltpu.async_copy(
                ptable_hbm.at[idx_v.at[j]],
                rows_v.at[pl.ds(j * _RPG, _RPG)],
                sem,
            )
            for j in range(_KC)
        ]
        for cp in copies:
            cp.wait()
        pltpu.sync_copy(rows_v, out_hbm.at[pl.ds(base * _RPG, _CHUNK)])
        return carry

    lax.fori_loop(0, _N_CHUNKS, body, 0)


def kernel(types, table):
    t = types.astype(jnp.int32).reshape(_BP, 2)
    pidx = (t[:, 0] * _NT + t[:, 1]).reshape(_BP // _RPG, _RPG)
    ptable = jnp.concatenate(
        [jnp.repeat(table, _NT, axis=0), jnp.tile(table, (_NT, 1))], axis=1
    )
    out = _embed_sc(pidx, ptable)
    return out.reshape(_BATCH, _HIST, _DIM)
